# baseline (speedup 1.0000x reference)
import functools

import jax
import jax.numpy as jnp
from jax import lax
from jax.experimental import pallas as pl
from jax.experimental.pallas import tpu as pltpu
from jax.experimental.pallas import tpu_sc as plsc

_N = 10000
_D = 128
_E = 320000

_NC = 2
_NS = 16
_CH = 128
_NCHUNK = _E // _CH
_KBIG = 80
_KSML = 78
_KSML_MAIN = 76
_RPT = 640
_AROWS = _NS * _RPT


def _sc_partial(x, edges, zeros):
    mesh = plsc.VectorSubcoreMesh(core_axis_name="c", subcore_axis_name="s")

    @functools.partial(
        pl.kernel,
        out_type=jax.ShapeDtypeStruct((_NC, _AROWS, _D), jnp.float32),
        mesh=mesh,
        scratch_types=[
            [pltpu.VMEM((_CH,), jnp.int32)] * 4,
            [pltpu.VMEM((_CH,), jnp.int32)] * 4,
            [pltpu.VMEM((_CH, _D), jnp.float32)] * 2,
            pltpu.VMEM_SHARED((_AROWS, _D), jnp.float32),
            [pltpu.SemaphoreType.DMA] * 4,
            [pltpu.SemaphoreType.DMA] * 4,
            [pltpu.SemaphoreType.DMA] * 2,
            [pltpu.SemaphoreType.DMA] * 2,
        ],
    )
    def k(x_hbm, e_hbm, z_hbm, part_hbm,
          cbufs, rbufs, bufs, acc, icsems, irsems, gsems, ssems):
        cid = lax.axis_index("c")
        sid = lax.axis_index("s")
        pltpu.sync_copy(z_hbm, acc.at[pl.ds(sid * _RPT, _RPT)])
        plsc.subcore_barrier()

        def ic_start(base, j, b):
            pltpu.async_copy(
                e_hbm.at[1].at[pl.ds((base + j) * _CH, _CH)],
                cbufs[b], icsems[b])

        def ic_wait(base, j, b):
            pltpu.make_async_copy(
                e_hbm.at[1].at[pl.ds((base + j) * _CH, _CH)],
                cbufs[b], icsems[b]).wait()

        def ir_start(base, j, b):
            pltpu.async_copy(
                e_hbm.at[0].at[pl.ds((base + j) * _CH, _CH)],
                rbufs[b], irsems[b])

        def ir_wait(base, j, b):
            pltpu.make_async_copy(
                e_hbm.at[0].at[pl.ds((base + j) * _CH, _CH)],
                rbufs[b], irsems[b]).wait()

        def g_start(b4, b2):
            pltpu.async_copy(x_hbm.at[cbufs[b4]], bufs[b2], gsems[b2])

        def g_wait(b4, b2):
            pltpu.make_async_copy(x_hbm.at[cbufs[b4]], bufs[b2],
                                  gsems[b2]).wait()

        def s_start(b4, b2):
            pltpu.async_copy(bufs[b2], acc.at[rbufs[b4]], ssems[b2],
                             add=True)

        def s_wait(b4, b2):
            pltpu.make_async_copy(bufs[b2], acc.at[rbufs[b4]],
                                  ssems[b2]).wait()

        def pipeline(base, kk):
            ic_start(base, 0, 0)
            ic_start(base, 1, 1)
            ir_start(base, 0, 0)

            def iter_body(j, b4, first, last):
                b2 = b4 % 2
                p4 = (b4 + 3) % 4
                q4 = (b4 + 2) % 4
                ic_wait(base, j, b4)
                if not first or b4 >= 2:
                    s_wait(q4, b2)
                g_start(b4, b2)
                if not first or b4 >= 1:
                    ir_wait(base, j - 1, p4)
                    g_wait(p4, 1 - b2)
                    s_start(p4, 1 - b2)
                if not last or b4 <= 1:
                    ic_start(base, j + 2, q4)
                if not last or b4 <= 2:
                    ir_start(base, j + 1, (b4 + 1) % 4)

            for b in range(4):
                iter_body(b, b, True, False)

            def group(jj, carry):
                for b in range(4):
                    iter_body(jj * 4 + b, b, False, False)
                return carry

            lax.fori_loop(1, kk // 4 - 1, group, 0)

            for b in range(4):
                iter_body(kk - 4 + b, b, False, True)
            ir_wait(base, kk - 1, 3)
            g_wait(3, 1)
            s_wait(2, 0)
            s_start(3, 1)
            s_wait(3, 1)

        def tail(base, j):
            pltpu.sync_copy(e_hbm.at[1].at[pl.ds((base + j) * _CH, _CH)],
                            cbufs[0])
            pltpu.sync_copy(e_hbm.at[0].at[pl.ds((base + j) * _CH, _CH)],
                            rbufs[0])
            g_start(0, 0)
            g_wait(0, 0)
            s_start(0, 0)
            s_wait(0, 0)

        half = cid * (_NCHUNK // _NC)

        @pl.when(sid < 1)
        def _():
            pipeline(half, _KBIG)

        @pl.when(sid >= 1)
        def _():
            base = half + _KBIG + (sid - 1) * _KSML
            pipeline(base, _KSML_MAIN)
            tail(base, _KSML_MAIN)
            tail(base, _KSML_MAIN + 1)

        plsc.subcore_barrier()
        pltpu.sync_copy(acc.at[pl.ds(sid * _RPT, _RPT)],
                        part_hbm.at[cid].at[pl.ds(sid * _RPT, _RPT)])

    return k(x, edges, zeros)


def _combine_body(eps_ref, x_ref, p0_ref, p1_ref, o_ref):
    o_ref[...] = ((1.0 + eps_ref[0, 0]) * x_ref[...]
                  + p0_ref[0] + p1_ref[0])


def _combine(x, part, eps):
    blk = 2000
    return pl.pallas_call(
        _combine_body,
        grid=(_N // blk,),
        in_specs=[
            pl.BlockSpec((1, 1), lambda i: (0, 0)),
            pl.BlockSpec((blk, _D), lambda i: (i, 0)),
            pl.BlockSpec((1, blk, _D), lambda i: (0, i, 0)),
            pl.BlockSpec((1, blk, _D), lambda i: (1, i, 0)),
        ],
        out_specs=pl.BlockSpec((blk, _D), lambda i: (i, 0)),
        out_shape=jax.ShapeDtypeStruct((_N, _D), jnp.float32),
    )(eps.reshape(1, 1), x, part, part)


def kernel(x, edge_index, eps):
    edges = edge_index.astype(jnp.int32)
    zeros = jnp.zeros((_RPT, _D), jnp.float32)
    part = _sc_partial(x, edges, zeros)
    return _combine(x, part, eps)


# speedup vs baseline: 1.0137x; 1.0137x over previous
import functools

import jax
import jax.numpy as jnp
from jax import lax
from jax.experimental import pallas as pl
from jax.experimental.pallas import tpu as pltpu
from jax.experimental.pallas import tpu_sc as plsc

_N = 10000
_D = 128
_E = 320000

_NC = 2
_NS = 16
_NW = _NC * _NS
_CH = 64
_NCHUNK = _E // _CH
_KBIG = 160
_KSML = 156
_RPT = 640
_AROWS = _NS * _RPT


def _sc_partial(x, edges, zeros):
    mesh = plsc.VectorSubcoreMesh(core_axis_name="c", subcore_axis_name="s")

    @functools.partial(
        pl.kernel,
        out_type=jax.ShapeDtypeStruct((_NC, _AROWS, _D), jnp.float32),
        mesh=mesh,
        scratch_types=[
            [pltpu.VMEM((_CH,), jnp.int32)] * 4,
            [pltpu.VMEM((_CH,), jnp.int32)] * 4,
            [pltpu.VMEM((_CH, _D), jnp.float32)] * 4,
            pltpu.VMEM_SHARED((_AROWS, _D), jnp.float32),
            [pltpu.SemaphoreType.DMA] * 4,
            [pltpu.SemaphoreType.DMA] * 4,
            [pltpu.SemaphoreType.DMA] * 4,
            [pltpu.SemaphoreType.DMA] * 4,
        ],
    )
    def k(x_hbm, e_hbm, z_hbm, part_hbm,
          cbufs, rbufs, bufs, acc, icsems, irsems, gsems, ssems):
        cid = lax.axis_index("c")
        sid = lax.axis_index("s")
        pltpu.sync_copy(z_hbm, acc.at[pl.ds(sid * _RPT, _RPT)])
        plsc.subcore_barrier()

        def pipeline(base, kk):
            def ic_start(j, b):
                pltpu.async_copy(
                    e_hbm.at[1].at[pl.ds((base + j) * _CH, _CH)],
                    cbufs[b], icsems[b])

            def ic_wait(j, b):
                pltpu.make_async_copy(
                    e_hbm.at[1].at[pl.ds((base + j) * _CH, _CH)],
                    cbufs[b], icsems[b]).wait()

            def ir_start(j, b):
                pltpu.async_copy(
                    e_hbm.at[0].at[pl.ds((base + j) * _CH, _CH)],
                    rbufs[b], irsems[b])

            def ir_wait(j, b):
                pltpu.make_async_copy(
                    e_hbm.at[0].at[pl.ds((base + j) * _CH, _CH)],
                    rbufs[b], irsems[b]).wait()

            def g_start(b):
                pltpu.async_copy(x_hbm.at[cbufs[b]], bufs[b], gsems[b])

            def g_wait(b):
                pltpu.make_async_copy(x_hbm.at[cbufs[b]], bufs[b],
                                      gsems[b]).wait()

            def s_start(b):
                pltpu.async_copy(bufs[b], acc.at[rbufs[b]], ssems[b],
                                 add=True)

            def s_wait(b):
                pltpu.make_async_copy(bufs[b], acc.at[rbufs[b]],
                                      ssems[b]).wait()

            ic_start(0, 0)
            ic_start(1, 1)
            ir_start(0, 0)

            def iter_body(j, b, first, last):
                b1, b2, b3 = (b + 1) % 4, (b + 2) % 4, (b + 3) % 4
                ic_wait(j, b)
                g_start(b)
                if not first or b >= 1:
                    ir_wait(j - 1, b3)
                    g_wait(b3)
                    s_start(b3)
                if not first or b >= 3:
                    s_wait(b1)
                if not last or b <= 1:
                    ic_start(j + 2, b2)
                if not last or b <= 2:
                    ir_start(j + 1, b1)

            for b in range(4):
                iter_body(b, b, True, False)

            def group(jj, carry):
                for b in range(4):
                    iter_body(jj * 4 + b, b, False, False)
                return carry

            lax.fori_loop(1, kk // 4 - 1, group, 0)

            for b in range(4):
                iter_body(kk - 4 + b, b, False, True)
            ir_wait(kk - 1, 3)
            g_wait(3)
            s_start(3)
            for b in (1, 2, 3):
                s_wait(b)

        half = cid * (_NCHUNK // _NC)

        @pl.when(sid < 1)
        def _():
            pipeline(half, _KBIG)

        @pl.when(sid >= 1)
        def _():
            pipeline(half + _KBIG + (sid - 1) * _KSML, _KSML)

        plsc.subcore_barrier()
        pltpu.sync_copy(acc.at[pl.ds(sid * _RPT, _RPT)],
                        part_hbm.at[cid].at[pl.ds(sid * _RPT, _RPT)])

    return k(x, edges, zeros)


def _combine_body(eps_ref, x_ref, p0_ref, p1_ref, o_ref):
    o_ref[...] = ((1.0 + eps_ref[0, 0]) * x_ref[...]
                  + p0_ref[0] + p1_ref[0])


def _combine(x, part, eps):
    blk = 2000
    return pl.pallas_call(
        _combine_body,
        grid=(_N // blk,),
        in_specs=[
            pl.BlockSpec((1, 1), lambda i: (0, 0)),
            pl.BlockSpec((blk, _D), lambda i: (i, 0)),
            pl.BlockSpec((1, blk, _D), lambda i: (0, i, 0)),
            pl.BlockSpec((1, blk, _D), lambda i: (1, i, 0)),
        ],
        out_specs=pl.BlockSpec((blk, _D), lambda i: (i, 0)),
        out_shape=jax.ShapeDtypeStruct((_N, _D), jnp.float32),
    )(eps.reshape(1, 1), x, part, part)


def kernel(x, edge_index, eps):
    edges = edge_index.astype(jnp.int32)
    zeros = jnp.zeros((_RPT, _D), jnp.float32)
    part = _sc_partial(x, edges, zeros)
    return _combine(x, part, eps)
